# baseline (speedup 1.0000x reference)
import functools

import jax
import jax.numpy as jnp
from jax import lax
from jax.experimental import pallas as pl
from jax.experimental.pallas import tpu as pltpu
from jax.experimental.pallas import tpu_sc as plsc

_NC = 2
_NS = 16
_NW = _NC * _NS
_C = 256
_NBUF = 2
_D = 1


@functools.lru_cache(maxsize=None)
def _make_gather(V, D, N, S):
    B = N * S
    rows_per_w = B // _NW
    K = rows_per_w // _C
    G = K // _NBUF
    cpj = N // _C
    assert B == _NW * K * _C and K % _NBUF == 0 and G >= 3 and _D < _NBUF
    assert N % _C == 0

    mesh = plsc.VectorSubcoreMesh(core_axis_name="c", subcore_axis_name="s")

    @functools.partial(
        pl.kernel,
        mesh=mesh,
        out_type=jax.ShapeDtypeStruct((S, N, D), jnp.float32),
        scratch_types=[
            pltpu.VMEM((rows_per_w,), jnp.int32),
            pltpu.VMEM((_NBUF, _C, D), jnp.float32),
        ] + [pltpu.SemaphoreType.DMA] * (2 * _NBUF),
    )
    def gather_k(table, idx, out, idx_v, rows_v, *sems):
        gsems = sems[:_NBUF]
        osems = sems[_NBUF:]
        w = lax.axis_index("s") * _NC + lax.axis_index("c")
        chunk0 = w * K
        pltpu.sync_copy(idx.at[pl.ds(chunk0 * _C, rows_per_w)], idx_v)

        def gcopy(c, b):
            return pltpu.make_async_copy(
                table.at[idx_v.at[pl.ds(c * _C, _C)]], rows_v.at[b],
                gsems[b])

        def ocopy(c, b):
            cg = chunk0 + c
            j = cg // cpj
            i0 = (cg % cpj) * _C
            return pltpu.make_async_copy(
                rows_v.at[b], out.at[j, pl.ds(i0, _C), :], osems[b])

        def step(c, b, wait_o, prefetch):
            gcopy(c, b).wait()
            ocopy(c, b).start()
            if prefetch:
                cf = c + _D
                bf = (b + _D) % _NBUF
                if wait_o:
                    ocopy(c, bf).wait()
                gcopy(cf, bf).start()

        for b in range(_D):
            gcopy(b, b).start()

        for b in range(_NBUF):
            step(b, b, wait_o=(b + _D >= _NBUF), prefetch=True)

        def body(g, carry):
            for b in range(_NBUF):
                step(g * _NBUF + b, b, wait_o=True, prefetch=True)
            return carry

        lax.fori_loop(1, G - 1, body, 0)

        for b in range(_NBUF):
            c = (G - 1) * _NBUF + b
            step(c, b, wait_o=True, prefetch=(b + _D < _NBUF))

        for b in range(_NBUF):
            ocopy(0, b).wait()

    return gather_k


def kernel(input, indices):
    V, D = input.shape
    N, S = indices.shape
    idx = indices.T.reshape(-1).astype(jnp.int32)
    out = _make_gather(V, D, N, S)(input, idx)
    return out.transpose(1, 0, 2)


# speedup vs baseline: 1.0333x; 1.0333x over previous
import functools

import jax
import jax.numpy as jnp
from jax import lax
from jax.experimental import pallas as pl
from jax.experimental.pallas import tpu as pltpu
from jax.experimental.pallas import tpu_sc as plsc

_NC = 2
_NS = 16
_NW = _NC * _NS
_C = 64
_NBUF = 8
_D = 6


@functools.lru_cache(maxsize=None)
def _make_gather(V, D, N, S):
    B = N * S
    rows_per_w = B // _NW
    K = rows_per_w // _C
    G = K // _NBUF
    cpj = N // _C
    assert B == _NW * K * _C and K % _NBUF == 0 and G >= 3 and _D < _NBUF
    assert N % _C == 0

    mesh = plsc.VectorSubcoreMesh(core_axis_name="c", subcore_axis_name="s")

    @functools.partial(
        pl.kernel,
        mesh=mesh,
        out_type=jax.ShapeDtypeStruct((S, N, D), jnp.float32),
        scratch_types=[
            pltpu.VMEM((rows_per_w,), jnp.int32),
            pltpu.VMEM((_NBUF, _C, D), jnp.float32),
        ] + [pltpu.SemaphoreType.DMA] * (2 * _NBUF),
    )
    def gather_k(table, idx, out, idx_v, rows_v, *sems):
        gsems = sems[:_NBUF]
        osems = sems[_NBUF:]
        w = lax.axis_index("s") * _NC + lax.axis_index("c")
        chunk0 = w * K
        pltpu.sync_copy(idx.at[pl.ds(chunk0 * _C, rows_per_w)], idx_v)

        def gcopy(c, b):
            return pltpu.make_async_copy(
                table.at[idx_v.at[pl.ds(c * _C, _C)]], rows_v.at[b],
                gsems[b])

        def ocopy(c, b):
            cg = chunk0 + c
            j = cg // cpj
            i0 = (cg % cpj) * _C
            return pltpu.make_async_copy(
                rows_v.at[b], out.at[j, pl.ds(i0, _C), :], osems[b])

        def step(c, b, wait_o, prefetch):
            gcopy(c, b).wait()
            ocopy(c, b).start()
            if prefetch:
                cf = c + _D
                bf = (b + _D) % _NBUF
                if wait_o:
                    ocopy(c, bf).wait()
                gcopy(cf, bf).start()

        for b in range(_D):
            gcopy(b, b).start()

        for b in range(_NBUF):
            step(b, b, wait_o=(b + _D >= _NBUF), prefetch=True)

        def body(g, carry):
            for b in range(_NBUF):
                step(g * _NBUF + b, b, wait_o=True, prefetch=True)
            return carry

        lax.fori_loop(1, G - 1, body, 0)

        for b in range(_NBUF):
            c = (G - 1) * _NBUF + b
            step(c, b, wait_o=True, prefetch=(b + _D < _NBUF))

        for b in range(_NBUF):
            ocopy(0, b).wait()

    return gather_k


def kernel(input, indices):
    V, D = input.shape
    N, S = indices.shape
    idx = indices.T.reshape(-1).astype(jnp.int32)
    out = _make_gather(V, D, N, S)(input, idx)
    return out.transpose(1, 0, 2)


# speedup vs baseline: 1.0373x; 1.0038x over previous
import functools

import jax
import jax.numpy as jnp
from jax import lax
from jax.experimental import pallas as pl
from jax.experimental.pallas import tpu as pltpu
from jax.experimental.pallas import tpu_sc as plsc

_NC = 2
_NS = 16
_NW = _NC * _NS
_C = 32
_NBUF = 8
_D = 6


@functools.lru_cache(maxsize=None)
def _make_gather(V, D, N, S):
    B = N * S
    rows_per_w = B // _NW
    K = rows_per_w // _C
    G = K // _NBUF
    cpj = N // _C
    assert B == _NW * K * _C and K % _NBUF == 0 and G >= 3 and _D < _NBUF
    assert N % _C == 0

    mesh = plsc.VectorSubcoreMesh(core_axis_name="c", subcore_axis_name="s")

    @functools.partial(
        pl.kernel,
        mesh=mesh,
        out_type=jax.ShapeDtypeStruct((S, N, D), jnp.float32),
        scratch_types=[
            pltpu.VMEM((rows_per_w,), jnp.int32),
            pltpu.VMEM((_NBUF, _C, D), jnp.float32),
        ] + [pltpu.SemaphoreType.DMA] * (2 * _NBUF),
    )
    def gather_k(table, idx, out, idx_v, rows_v, *sems):
        gsems = sems[:_NBUF]
        osems = sems[_NBUF:]
        w = lax.axis_index("s") * _NC + lax.axis_index("c")
        chunk0 = w * K
        pltpu.sync_copy(idx.at[pl.ds(chunk0 * _C, rows_per_w)], idx_v)

        def gcopy(c, b):
            return pltpu.make_async_copy(
                table.at[idx_v.at[pl.ds(c * _C, _C)]], rows_v.at[b],
                gsems[b])

        def ocopy(c, b):
            cg = chunk0 + c
            j = cg // cpj
            i0 = (cg % cpj) * _C
            return pltpu.make_async_copy(
                rows_v.at[b], out.at[j, pl.ds(i0, _C), :], osems[b])

        def step(c, b, wait_o, prefetch):
            gcopy(c, b).wait()
            ocopy(c, b).start()
            if prefetch:
                cf = c + _D
                bf = (b + _D) % _NBUF
                if wait_o:
                    ocopy(c, bf).wait()
                gcopy(cf, bf).start()

        for b in range(_D):
            gcopy(b, b).start()

        for b in range(_NBUF):
            step(b, b, wait_o=(b + _D >= _NBUF), prefetch=True)

        def body(g, carry):
            for b in range(_NBUF):
                step(g * _NBUF + b, b, wait_o=True, prefetch=True)
            return carry

        lax.fori_loop(1, G - 1, body, 0)

        for b in range(_NBUF):
            c = (G - 1) * _NBUF + b
            step(c, b, wait_o=True, prefetch=(b + _D < _NBUF))

        for b in range(_NBUF):
            ocopy(0, b).wait()

    return gather_k


def kernel(input, indices):
    V, D = input.shape
    N, S = indices.shape
    idx = indices.T.reshape(-1).astype(jnp.int32)
    out = _make_gather(V, D, N, S)(input, idx)
    return out.transpose(1, 0, 2)
